# baseline (speedup 1.0000x reference)
import functools

import jax
import jax.numpy as jnp
from jax import lax
from jax.experimental import pallas as pl
from jax.experimental.pallas import tpu as pltpu
from jax.experimental.pallas import tpu_sc as plsc

LOG2E = 1.4426950408889634

_R1, _C1 = 512, 2048
_R2, _C2 = 512, 2048


def _p1_body(zn_ref, cn_ref, z_ref, cb_ref, m_ref, r_ref, idx_ref, summ_ref,
             fidx_ref, s_ref,
             *, n_i, n_j, c_blk, k_total):
    i = pl.program_id(0)
    j = pl.program_id(1)
    t = lax.dot_general(z_ref[...], cb_ref[...],
                        dimension_numbers=(((1,), (1,)), ((), ())),
                        preferred_element_type=jnp.float32)
    d = (zn_ref[...] + t) + cn_ref[...]
    lmin = jnp.min(d, axis=1, keepdims=True)
    g = (lmin - d) * LOG2E
    jg = (lax.broadcasted_iota(jnp.int32, d.shape, 1).astype(jnp.float32)
          + jnp.float32(j * c_blk))
    larg = jnp.min(jnp.where(g == 0.0, jg, jnp.float32(2 * k_total)),
                   axis=1, keepdims=True)
    ones_c = jnp.ones((g.shape[1], 1), jnp.float32)
    ls = lax.dot_general(jnp.exp2(g), ones_c,
                         dimension_numbers=(((1,), (0,)), ((), ())),
                         preferred_element_type=jnp.float32)

    @pl.when(j == 0)
    def _():
        m_ref[...] = lmin
        fidx_ref[...] = larg
        s_ref[...] = ls

    @pl.when(j > 0)
    def _():
        mo = m_ref[...]
        so = s_ref[...]
        mn = jnp.minimum(mo, lmin)
        s_ref[...] = (so * jnp.exp2((mn - mo) * LOG2E)
                      + ls * jnp.exp2((mn - lmin) * LOG2E))
        m_ref[...] = mn
        fidx_ref[...] = jnp.where(lmin < mo, larg, fidx_ref[...])

    @pl.when(j == n_j - 1)
    def _():
        part = jnp.sum(m_ref[...])
        idx_ref[...] = fidx_ref[...].astype(jnp.int32)
        s_here = s_ref[...]
        r_ref[...] = ((m_ref[...] - zn_ref[...])
                      - jnp.log(s_here)) * LOG2E

        @pl.when(i == 0)
        def _():
            summ_ref[0, 0] = part

        @pl.when(i > 0)
        def _():
            summ_ref[0, 0] = summ_ref[0, 0] + part


def _p2_body(cn_ref, mr_ref, z_ref, cb_ref, ent_ref, acc_ref,
             *, n_i, n_j, n_rows):
    j = pl.program_id(0)
    i = pl.program_id(1)
    t = lax.dot_general(z_ref[...], cb_ref[...],
                        dimension_numbers=(((1,), (1,)), ((), ())),
                        preferred_element_type=jnp.float32)
    w = jnp.exp2(mr_ref[...] - (t + cn_ref[...]))
    ones_r = jnp.ones((1, w.shape[0]), jnp.float32)
    cs = lax.dot_general(ones_r, w,
                         dimension_numbers=(((1,), (0,)), ((), ())),
                         preferred_element_type=jnp.float32)

    @pl.when(i == 0)
    def _():
        acc_ref[...] = cs

    @pl.when(i > 0)
    def _():
        acc_ref[...] = acc_ref[...] + cs

    @pl.when(i == n_i - 1)
    def _():
        p = acc_ref[...] * (1.0 / n_rows)
        part = -jnp.sum(p * jnp.log(p + 1e-10))

        @pl.when(j == 0)
        def _():
            ent_ref[0, 0] = part

        @pl.when(j > 0)
        def _():
            ent_ref[0, 0] = ent_ref[0, 0] + part


def _make_sc_gather(n_rows, d_model, n_workers, chunk):
    b_per_w = n_rows // n_workers
    n_chunks = b_per_w // chunk
    mesh = plsc.VectorSubcoreMesh(core_axis_name="c", subcore_axis_name="s")

    @functools.partial(
        pl.kernel, mesh=mesh,
        out_type=jax.ShapeDtypeStruct((n_rows, d_model), jnp.float32),
        scratch_types=[
            pltpu.VMEM((b_per_w,), jnp.int32),
            pltpu.VMEM((chunk, d_model), jnp.float32),
            pltpu.VMEM((chunk, d_model), jnp.float32),
            pltpu.SemaphoreType.DMA,
            pltpu.SemaphoreType.DMA,
        ],
    )
    def gather_k(cb_hbm, idx_hbm, out_hbm, idx_v, rows_a, rows_b, sem_a, sem_b):
        wid = lax.axis_index("s") * 2 + lax.axis_index("c")
        base = wid * b_per_w
        pltpu.sync_copy(idx_hbm.at[pl.ds(base, b_per_w)], idx_v)
        bufs = ((rows_a, sem_a), (rows_b, sem_b))
        cps = []
        for c in range(n_chunks):
            buf, sem = bufs[c % 2]
            cps.append(pltpu.async_copy(
                cb_hbm.at[idx_v.at[pl.ds(c * chunk, chunk)]], buf, sem))
            if c >= 1:
                cps[c - 1].wait()
                pbuf, _ = bufs[(c - 1) % 2]
                pltpu.sync_copy(
                    pbuf, out_hbm.at[pl.ds(base + (c - 1) * chunk, chunk)])
        cps[n_chunks - 1].wait()
        lbuf, _ = bufs[(n_chunks - 1) % 2]
        pltpu.sync_copy(
            lbuf, out_hbm.at[pl.ds(base + (n_chunks - 1) * chunk, chunk)])

    return gather_k


def kernel(z, codebook):
    b, k_seq, d_model = z.shape
    n = b * k_seq
    kcb = codebook.shape[0]
    z2 = z.reshape(n, d_model)

    zn = jnp.sum(z2 ** 2, axis=1, keepdims=True)
    cn = jnp.sum(codebook ** 2, axis=1)[None, :]

    za = z2 * (-2.0)
    n_i1, n_j1 = n // _R1, kcb // _C1
    grid1 = (n_i1, n_j1)
    m, mr, idx, summ = pl.pallas_call(
        functools.partial(_p1_body, n_i=n_i1, n_j=n_j1, c_blk=_C1,
                          k_total=kcb),
        grid=grid1,
        in_specs=[
            pl.BlockSpec((_R1, 1), lambda i, j: (i, 0)),
            pl.BlockSpec((1, _C1), lambda i, j: (0, j)),
            pl.BlockSpec((_R1, d_model), lambda i, j: (i, 0)),
            pl.BlockSpec((_C1, d_model), lambda i, j: (j, 0)),
        ],
        out_specs=[
            pl.BlockSpec((_R1, 1), lambda i, j: (i, 0)),
            pl.BlockSpec((_R1, 1), lambda i, j: (i, 0)),
            pl.BlockSpec((_R1, 1), lambda i, j: (i, 0)),
            pl.BlockSpec((1, 1), lambda i, j: (0, 0),
                         memory_space=pltpu.SMEM),
        ],
        out_shape=[
            jax.ShapeDtypeStruct((n, 1), jnp.float32),
            jax.ShapeDtypeStruct((n, 1), jnp.float32),
            jax.ShapeDtypeStruct((n, 1), jnp.int32),
            jax.ShapeDtypeStruct((1, 1), jnp.float32),
        ],
        scratch_shapes=[
            pltpu.VMEM((_R1, 1), jnp.float32),
            pltpu.VMEM((_R1, 1), jnp.float32),
        ],
    )(zn, cn, za, codebook)

    gather_k = _make_sc_gather(n, d_model, 32, 128)
    z_q = gather_k(codebook, idx.reshape(n))

    za2b = (z2 * (-2.0 * LOG2E)).astype(jnp.bfloat16)
    cbb = codebook.astype(jnp.bfloat16)
    cnl2 = cn * LOG2E
    n_i2, n_j2 = n // _R2, kcb // _C2
    grid2 = (n_j2, n_i2)
    ent = pl.pallas_call(
        functools.partial(_p2_body, n_i=n_i2, n_j=n_j2, n_rows=n),
        grid=grid2,
        in_specs=[
            pl.BlockSpec((1, _C2), lambda j, i: (0, j)),
            pl.BlockSpec((_R2, 1), lambda j, i: (i, 0)),
            pl.BlockSpec((_R2, d_model), lambda j, i: (i, 0)),
            pl.BlockSpec((_C2, d_model), lambda j, i: (j, 0)),
        ],
        out_specs=pl.BlockSpec((1, 1), lambda j, i: (0, 0),
                               memory_space=pltpu.SMEM),
        out_shape=jax.ShapeDtypeStruct((1, 1), jnp.float32),
        scratch_shapes=[pltpu.VMEM((1, _C2), jnp.float32)],
    )(cnl2, mr, za2b, cbb)

    sum_min = summ[0, 0]
    entropy = ent[0, 0]
    max_ent = jnp.log(jnp.float32(kcb))
    total_loss = (1.25 * sum_min / jnp.float32(n * d_model)
                  + 0.1 * (max_ent - entropy) / max_ent)
    return (z_q.reshape(b, k_seq, d_model), total_loss,
            idx.reshape(b, k_seq))


# speedup vs baseline: 1.4053x; 1.4053x over previous
import functools

import jax
import jax.numpy as jnp
from jax import lax
from jax.experimental import pallas as pl
from jax.experimental.pallas import tpu as pltpu
from jax.experimental.pallas import tpu_sc as plsc

LOG2E = 1.4426950408889634

_R1, _C1 = 512, 2048
_R2, _C2 = 512, 2048


def _p1_body(zn_ref, cn_ref, z_ref, cb_ref, m_ref, r_ref, idx_ref, summ_ref,
             fidx_ref, s_ref,
             *, n_i, n_j, c_blk, k_total):
    i = pl.program_id(0)
    j = pl.program_id(1)
    t = lax.dot_general(z_ref[...], cb_ref[...],
                        dimension_numbers=(((1,), (1,)), ((), ())),
                        preferred_element_type=jnp.float32)
    d = (zn_ref[...] + t) + cn_ref[...]
    lmin = jnp.min(d, axis=1, keepdims=True)
    g = (lmin - d) * LOG2E
    jg = (lax.broadcasted_iota(jnp.int32, d.shape, 1).astype(jnp.float32)
          + jnp.float32(j * c_blk))
    larg = jnp.min(jnp.where(g == 0.0, jg, jnp.float32(2 * k_total)),
                   axis=1, keepdims=True)
    ls = jnp.sum(jnp.exp2(g), axis=1, keepdims=True)

    @pl.when(j == 0)
    def _():
        m_ref[...] = lmin
        fidx_ref[...] = larg
        s_ref[...] = ls

    @pl.when(j > 0)
    def _():
        mo = m_ref[...]
        so = s_ref[...]
        mn = jnp.minimum(mo, lmin)
        s_ref[...] = (so * jnp.exp2((mn - mo) * LOG2E)
                      + ls * jnp.exp2((mn - lmin) * LOG2E))
        m_ref[...] = mn
        fidx_ref[...] = jnp.where(lmin < mo, larg, fidx_ref[...])

    @pl.when(j == n_j - 1)
    def _():
        part = jnp.sum(m_ref[...])
        idx_ref[...] = fidx_ref[...].astype(jnp.int32)
        s_here = s_ref[...]
        r_ref[...] = ((m_ref[...] - zn_ref[...])
                      - jnp.log(s_here)) * LOG2E

        @pl.when(i == 0)
        def _():
            summ_ref[0, 0] = part

        @pl.when(i > 0)
        def _():
            summ_ref[0, 0] = summ_ref[0, 0] + part


def _p2_body(cn_ref, mr_ref, z_ref, cb_ref, ent_ref, acc_ref,
             *, n_i, n_j, n_rows):
    j = pl.program_id(0)
    i = pl.program_id(1)
    t = lax.dot_general(z_ref[...], cb_ref[...],
                        dimension_numbers=(((1,), (1,)), ((), ())),
                        preferred_element_type=jnp.float32)
    w = jnp.exp2(mr_ref[...] - (t + cn_ref[...]))
    cs = jnp.sum(w, axis=0, keepdims=True)

    @pl.when(i == 0)
    def _():
        acc_ref[...] = cs

    @pl.when(i > 0)
    def _():
        acc_ref[...] = acc_ref[...] + cs

    @pl.when(i == n_i - 1)
    def _():
        p = acc_ref[...] * (1.0 / n_rows)
        part = -jnp.sum(p * jnp.log(p + 1e-10))

        @pl.when(j == 0)
        def _():
            ent_ref[0, 0] = part

        @pl.when(j > 0)
        def _():
            ent_ref[0, 0] = ent_ref[0, 0] + part


def _make_sc_gather(n_rows, d_model, n_workers, chunk):
    b_per_w = n_rows // n_workers
    n_chunks = b_per_w // chunk
    mesh = plsc.VectorSubcoreMesh(core_axis_name="c", subcore_axis_name="s")

    @functools.partial(
        pl.kernel, mesh=mesh,
        out_type=jax.ShapeDtypeStruct((n_rows, d_model), jnp.float32),
        scratch_types=[
            pltpu.VMEM((b_per_w,), jnp.int32),
            pltpu.VMEM((chunk, d_model), jnp.float32),
            pltpu.VMEM((chunk, d_model), jnp.float32),
            pltpu.SemaphoreType.DMA,
            pltpu.SemaphoreType.DMA,
        ],
    )
    def gather_k(cb_hbm, idx_hbm, out_hbm, idx_v, rows_a, rows_b, sem_a, sem_b):
        wid = lax.axis_index("s") * 2 + lax.axis_index("c")
        base = wid * b_per_w
        pltpu.sync_copy(idx_hbm.at[pl.ds(base, b_per_w)], idx_v)
        bufs = ((rows_a, sem_a), (rows_b, sem_b))
        cps = []
        for c in range(n_chunks):
            buf, sem = bufs[c % 2]
            cps.append(pltpu.async_copy(
                cb_hbm.at[idx_v.at[pl.ds(c * chunk, chunk)]], buf, sem))
            if c >= 1:
                cps[c - 1].wait()
                pbuf, _ = bufs[(c - 1) % 2]
                pltpu.sync_copy(
                    pbuf, out_hbm.at[pl.ds(base + (c - 1) * chunk, chunk)])
        cps[n_chunks - 1].wait()
        lbuf, _ = bufs[(n_chunks - 1) % 2]
        pltpu.sync_copy(
            lbuf, out_hbm.at[pl.ds(base + (n_chunks - 1) * chunk, chunk)])

    return gather_k


def kernel(z, codebook):
    b, k_seq, d_model = z.shape
    n = b * k_seq
    kcb = codebook.shape[0]
    z2 = z.reshape(n, d_model)

    zn = jnp.sum(z2 ** 2, axis=1, keepdims=True)
    cn = jnp.sum(codebook ** 2, axis=1)[None, :]

    za = z2 * (-2.0)
    n_i1, n_j1 = n // _R1, kcb // _C1
    grid1 = (n_i1, n_j1)
    m, mr, idx, summ = pl.pallas_call(
        functools.partial(_p1_body, n_i=n_i1, n_j=n_j1, c_blk=_C1,
                          k_total=kcb),
        grid=grid1,
        in_specs=[
            pl.BlockSpec((_R1, 1), lambda i, j: (i, 0)),
            pl.BlockSpec((1, _C1), lambda i, j: (0, j)),
            pl.BlockSpec((_R1, d_model), lambda i, j: (i, 0)),
            pl.BlockSpec((_C1, d_model), lambda i, j: (j, 0)),
        ],
        out_specs=[
            pl.BlockSpec((_R1, 1), lambda i, j: (i, 0)),
            pl.BlockSpec((_R1, 1), lambda i, j: (i, 0)),
            pl.BlockSpec((_R1, 1), lambda i, j: (i, 0)),
            pl.BlockSpec((1, 1), lambda i, j: (0, 0),
                         memory_space=pltpu.SMEM),
        ],
        out_shape=[
            jax.ShapeDtypeStruct((n, 1), jnp.float32),
            jax.ShapeDtypeStruct((n, 1), jnp.float32),
            jax.ShapeDtypeStruct((n, 1), jnp.int32),
            jax.ShapeDtypeStruct((1, 1), jnp.float32),
        ],
        scratch_shapes=[
            pltpu.VMEM((_R1, 1), jnp.float32),
            pltpu.VMEM((_R1, 1), jnp.float32),
        ],
    )(zn, cn, za, codebook)

    gather_k = _make_sc_gather(n, d_model, 32, 128)
    z_q = gather_k(codebook, idx.reshape(n))

    za2b = (z2 * (-2.0 * LOG2E)).astype(jnp.bfloat16)
    cbb = codebook.astype(jnp.bfloat16)
    cnl2 = cn * LOG2E
    n_i2, n_j2 = n // _R2, kcb // _C2
    grid2 = (n_j2, n_i2)
    ent = pl.pallas_call(
        functools.partial(_p2_body, n_i=n_i2, n_j=n_j2, n_rows=n),
        grid=grid2,
        in_specs=[
            pl.BlockSpec((1, _C2), lambda j, i: (0, j)),
            pl.BlockSpec((_R2, 1), lambda j, i: (i, 0)),
            pl.BlockSpec((_R2, d_model), lambda j, i: (i, 0)),
            pl.BlockSpec((_C2, d_model), lambda j, i: (j, 0)),
        ],
        out_specs=pl.BlockSpec((1, 1), lambda j, i: (0, 0),
                               memory_space=pltpu.SMEM),
        out_shape=jax.ShapeDtypeStruct((1, 1), jnp.float32),
        scratch_shapes=[pltpu.VMEM((1, _C2), jnp.float32)],
    )(cnl2, mr, za2b, cbb)

    sum_min = summ[0, 0]
    entropy = ent[0, 0]
    max_ent = jnp.log(jnp.float32(kcb))
    total_loss = (1.25 * sum_min / jnp.float32(n * d_model)
                  + 0.1 * (max_ent - entropy) / max_ent)
    return (z_q.reshape(b, k_seq, d_model), total_loss,
            idx.reshape(b, k_seq))
